# baseline (speedup 1.0000x reference)
import jax
import jax.numpy as jnp
from jax import lax
from jax.experimental import pallas as pl
from jax.experimental.pallas import tpu as pltpu
from jax.experimental.pallas import tpu_sc as plsc

VOCAB = 1_000_000
D = 64
DP = 128
B_TOTAL = 16384 * 50
CHUNK = 128
NC, NS = 2, 16
NW = NC * NS
BPW = B_TOTAL // NW
NCH = BPW // CHUNK
NBUF = 5
T = NCH // NBUF


def _body(idx_hbm, table_hbm, out_hbm, idx_v, buf_v, gsem, ssem):
    wid = lax.axis_index("s") * NC + lax.axis_index("c")
    pltpu.sync_copy(idx_hbm.at[pl.ds(wid * NCH, NCH)], idx_v)
    base = wid * BPW

    def fire_gathers(t, drain_prev):
        descs = []
        for b in range(NBUF):
            j = t * NBUF + b
            if drain_prev:
                pltpu.make_async_copy(buf_v.at[b, :, pl.ds(0, D)],
                                      out_hbm.at[pl.ds(base, CHUNK)],
                                      ssem.at[b]).wait()
            descs.append(
                pltpu.async_copy(table_hbm.at[idx_v.at[j]], buf_v.at[b],
                                 gsem.at[b]))
        return descs

    def drain_and_scatter(t, gdescs):
        for b in range(NBUF):
            j = t * NBUF + b
            gdescs[b].wait()
            pltpu.async_copy(buf_v.at[b, :, pl.ds(0, D)],
                             out_hbm.at[pl.ds(base + j * CHUNK, CHUNK)],
                             ssem.at[b])

    gdescs = fire_gathers(0, drain_prev=False)
    drain_and_scatter(0, gdescs)

    @pl.loop(1, T)
    def _cycle(t):
        gd = fire_gathers(t, drain_prev=True)
        drain_and_scatter(t, gd)

    for b in range(NBUF):
        pltpu.make_async_copy(buf_v.at[b, :, pl.ds(0, D)],
                              out_hbm.at[pl.ds(base, CHUNK)],
                              ssem.at[b]).wait()


@jax.jit
def _embed(idx2d, table_p):
    mesh = plsc.VectorSubcoreMesh(core_axis_name="c", subcore_axis_name="s")
    return pl.kernel(
        _body,
        out_type=jax.ShapeDtypeStruct((B_TOTAL, D), jnp.float32),
        mesh=mesh,
        scratch_types=[
            pltpu.VMEM((NCH, CHUNK), jnp.int32),
            pltpu.VMEM((NBUF, CHUNK, DP), jnp.float32),
            pltpu.SemaphoreType.DMA((NBUF,)),
            pltpu.SemaphoreType.DMA((NBUF,)),
        ],
        compiler_params=pltpu.CompilerParams(use_tc_tiling_on_sc=False),
    )(idx2d, table_p)


def kernel(token_ids, weight):
    idx2d = token_ids.reshape(-1).astype(jnp.int32).reshape(NW * NCH, CHUNK)
    table_p = jnp.pad(weight, ((0, 0), (0, DP - D)))
    out = _embed(idx2d, table_p)
    return out.reshape(token_ids.shape[0], token_ids.shape[1], D)


# speedup vs baseline: 1.4184x; 1.4184x over previous
import jax
import jax.numpy as jnp
from jax import lax
from jax.experimental import pallas as pl
from jax.experimental.pallas import tpu as pltpu
from jax.experimental.pallas import tpu_sc as plsc

VOCAB = 1_000_000
D = 64
B = 16384
S = 50
SPAD = 56
DPAD = 128
NC, NS = 2, 16
NW = NC * NS
RPW = B // NW
NBUF = 8
T = RPW // NBUF


def _body(idx_hbm, table_hbm, out_hbm, idx_v, buf_v, gsem, ssem):
    wid = lax.axis_index("s") * NC + lax.axis_index("c")
    pltpu.sync_copy(idx_hbm.at[pl.ds(wid * RPW, RPW)], idx_v)
    base = wid * RPW

    def gather_row(j, b):
        return pltpu.async_copy(
            table_hbm.at[idx_v.at[j]], buf_v.at[b], gsem.at[b])

    def scatter_row(j, b):
        pltpu.async_copy(
            buf_v.at[b],
            out_hbm.at[pl.ds((base + j) * SPAD, S), pl.ds(0, D)],
            ssem.at[b])

    def fire_gathers(t, drain_prev):
        descs = []
        for b in range(NBUF):
            if drain_prev:
                pltpu.make_async_copy(
                    buf_v.at[b],
                    out_hbm.at[pl.ds(base * SPAD, S), pl.ds(0, D)],
                    ssem.at[b]).wait()
            descs.append(gather_row(t * NBUF + b, b))
        return descs

    def drain_and_scatter(t, gdescs):
        for b in range(NBUF):
            gdescs[b].wait()
            scatter_row(t * NBUF + b, b)

    gdescs = fire_gathers(0, drain_prev=False)
    drain_and_scatter(0, gdescs)

    @pl.loop(1, T)
    def _cycle(t):
        gd = fire_gathers(t, drain_prev=True)
        drain_and_scatter(t, gd)

    for b in range(NBUF):
        pltpu.make_async_copy(
            buf_v.at[b],
            out_hbm.at[pl.ds(base * SPAD, S), pl.ds(0, D)],
            ssem.at[b]).wait()


@jax.jit
def _embed(token_ids, weight):
    mesh = plsc.VectorSubcoreMesh(core_axis_name="c", subcore_axis_name="s")
    return pl.kernel(
        _body,
        out_type=jax.ShapeDtypeStruct((B * SPAD, DPAD), jnp.float32),
        mesh=mesh,
        scratch_types=[
            pltpu.VMEM((RPW, S), jnp.int32),
            pltpu.VMEM((NBUF, S, D), jnp.float32),
            pltpu.SemaphoreType.DMA((NBUF,)),
            pltpu.SemaphoreType.DMA((NBUF,)),
        ],
        compiler_params=pltpu.CompilerParams(use_tc_tiling_on_sc=False),
    )(token_ids, weight)


def kernel(token_ids, weight):
    if token_ids.dtype != jnp.int32:
        token_ids = token_ids.astype(jnp.int32)
    out_p = _embed(token_ids, weight)
    return out_p.reshape(B, SPAD, DPAD)[:, :S, :D]
